# Initial kernel scaffold; baseline (speedup 1.0000x reference)
import jax
import jax.numpy as jnp
from jax.experimental import pallas as pl


def kernel(states, Wg, Wi, bi, Ws, bs, Wo, bo):
    raise NotImplementedError("write your pallas kernel here")


# speedup vs baseline: 2.9778x; 2.9778x over previous
import functools

import jax
import jax.numpy as jnp
from jax import lax
from jax.experimental import pallas as pl
from jax.experimental.pallas import tpu as pltpu
from jax.experimental.pallas import tpu_sc as plsc

D = 768
F = 3072
E = 8
T = 2048
TILE_R = 256
NT = 24
NR = NT * TILE_R

_NC, _NS = 2, 16
_NW = _NC * _NS
_CHUNK = T // _NW
_LANES = 16


def _router_body(x_ref, wg_ref, misc_ref, meta_ref, strict_ref, oh_ref):
    x = x_ref[...]
    logits = jnp.dot(x, wg_ref[...], preferred_element_type=jnp.float32)

    lane = lax.broadcasted_iota(jnp.int32, (T, 128), 1)
    validl = lane < E
    l = jnp.where(validl, logits, -1e30)

    m1 = jnp.max(l, axis=1, keepdims=True)
    i1 = jnp.min(jnp.where((l == m1) & validl, lane, 999), axis=1,
                 keepdims=True)
    l2 = jnp.where(lane == i1, -1e30, l)
    m2 = jnp.max(l2, axis=1, keepdims=True)
    i2 = jnp.min(jnp.where((l2 == m2) & validl & (lane != i1), lane, 999),
                 axis=1, keepdims=True)
    s = jnp.exp(m2 - m1)
    w0 = 1.0 / (1.0 + s)
    w1 = s * w0

    oh0 = (lane == i1).astype(jnp.float32)
    oh1 = (lane == i2).astype(jnp.float32)
    oh_ref[...] = oh0 + oh1

    tri = (lax.broadcasted_iota(jnp.int32, (128, 128), 0) >
           lax.broadcasted_iota(jnp.int32, (128, 128), 1)
           ).astype(jnp.float32)

    def chunk(j, carry):
        blk = oh_ref[pl.ds(j * 128, 128), :]
        strict_ref[pl.ds(j * 128, 128), :] = (
            jnp.dot(tri, blk, preferred_element_type=jnp.float32) + carry)
        return carry + jnp.sum(blk, axis=0, keepdims=True)

    cnt = lax.fori_loop(0, T // 128, chunk,
                        jnp.zeros((1, 128), jnp.float32))

    cnt_i = jnp.round(cnt).astype(jnp.int32)
    pc_i = ((cnt_i + (TILE_R - 1)) // TILE_R) * TILE_R
    triu = (lax.broadcasted_iota(jnp.int32, (128, 128), 0) <
            lax.broadcasted_iota(jnp.int32, (128, 128), 1)
            ).astype(jnp.float32)
    offs_f = jnp.dot(pc_i.astype(jnp.float32), triu,
                     preferred_element_type=jnp.float32)
    offs_i = jnp.round(offs_f).astype(jnp.int32)

    strict = strict_ref[...]
    pos0 = jnp.sum((offs_f + strict) * oh0, axis=1, keepdims=True)
    pos1 = jnp.sum((offs_f + strict) * oh1, axis=1, keepdims=True)

    misc_ref[...] = jnp.where(
        lane == 0, w0,
        jnp.where(lane == 1, w1,
                  jnp.where(lane == 2, pos0,
                            jnp.where(lane == 3, pos1, 0.0))))

    tile_end = (offs_i + pc_i) // TILE_R
    j2 = lax.broadcasted_iota(jnp.int32, (128, 128), 0)
    e2 = lax.broadcasted_iota(jnp.int32, (128, 128), 1)
    ge = ((j2 >= tile_end) & (e2 < E)).astype(jnp.int32)
    texp = jnp.minimum(jnp.sum(ge, axis=1, keepdims=True), E - 1)
    total_tiles = jnp.max(tile_end, axis=1, keepdims=True)
    jcol = lax.broadcasted_iota(jnp.int32, (128, 1), 0)
    tvalid = (jcol < total_tiles).astype(jnp.float32)
    meta_ref[...] = jnp.where(lane[:128, :] == 0, texp.astype(jnp.float32),
                              jnp.where(lane[:128, :] == 1, tvalid, 0.0))


def _run_router(x, wg_pad):
    return pl.pallas_call(
        _router_body,
        out_shape=(jax.ShapeDtypeStruct((T, 128), jnp.float32),
                   jax.ShapeDtypeStruct((128, 128), jnp.float32)),
        in_specs=[pl.BlockSpec((T, D), lambda: (0, 0)),
                  pl.BlockSpec((D, 128), lambda: (0, 0))],
        out_specs=(pl.BlockSpec((T, 128), lambda: (0, 0)),
                   pl.BlockSpec((128, 128), lambda: (0, 0))),
        scratch_shapes=[pltpu.VMEM((T, 128), jnp.float32),
                        pltpu.VMEM((T, 128), jnp.float32)],
    )(x, wg_pad)


def _dispatch_body(x_hbm, pos0_hbm, pos1_hbm, xs_hbm,
                   idx0_v, idx1_v, rows_v, sem):
    wid = lax.axis_index("s") * _NC + lax.axis_index("c")
    base = wid * _CHUNK
    pltpu.sync_copy(pos0_hbm.at[pl.ds(base, _CHUNK)], idx0_v)
    pltpu.sync_copy(pos1_hbm.at[pl.ds(base, _CHUNK)], idx1_v)
    pltpu.sync_copy(x_hbm.at[pl.ds(base, _CHUNK)], rows_v)
    pltpu.async_copy(rows_v, xs_hbm.at[idx0_v], sem).wait()
    pltpu.async_copy(rows_v, xs_hbm.at[idx1_v], sem).wait()


def _run_dispatch(x, pos0, pos1):
    mesh = plsc.VectorSubcoreMesh(core_axis_name="c", subcore_axis_name="s")
    return pl.kernel(
        _dispatch_body,
        out_type=jax.ShapeDtypeStruct((NR, D), jnp.float32),
        mesh=mesh,
        scratch_types=[
            pltpu.VMEM((_CHUNK,), jnp.int32),
            pltpu.VMEM((_CHUNK,), jnp.int32),
            pltpu.VMEM((_CHUNK, D), jnp.float32),
            pltpu.SemaphoreType.DMA,
        ],
    )(x, pos0, pos1)


def _mlp_body(texp_ref, tvalid_ref, xs_ref, wi_ref, bi_ref, ws_ref, bs_ref,
              wo_ref, bo_ref, y_ref):
    i = pl.program_id(0)

    @pl.when(tvalid_ref[i] == 1)
    def _():
        xb = xs_ref[...].astype(jnp.bfloat16)
        h = jnp.dot(xb, wi_ref[0], preferred_element_type=jnp.float32)
        h = h + bi_ref[0]
        g = 0.5 * h * (1.0 + lax.erf(h * 0.7071067811865476))
        sc = jnp.dot(xb, ws_ref[0], preferred_element_type=jnp.float32)
        sc = sc + bs_ref[0]
        inner = (g * sc).astype(jnp.bfloat16)
        y = jnp.dot(inner, wo_ref[0], preferred_element_type=jnp.float32)
        y_ref[...] = y + bo_ref[0]


def _run_mlp(texp, tvalid, xs, wi_b, bi, ws_b, bs, wo_b, bo):
    grid_spec = pltpu.PrefetchScalarGridSpec(
        num_scalar_prefetch=2,
        grid=(NT,),
        in_specs=[
            pl.BlockSpec((TILE_R, D), lambda i, te, tv: (i, 0)),
            pl.BlockSpec((1, D, F), lambda i, te, tv: (te[i], 0, 0)),
            pl.BlockSpec((1, 1, F), lambda i, te, tv: (te[i], 0, 0)),
            pl.BlockSpec((1, D, F), lambda i, te, tv: (te[i], 0, 0)),
            pl.BlockSpec((1, 1, F), lambda i, te, tv: (te[i], 0, 0)),
            pl.BlockSpec((1, F, D), lambda i, te, tv: (te[i], 0, 0)),
            pl.BlockSpec((1, 1, D), lambda i, te, tv: (te[i], 0, 0)),
        ],
        out_specs=pl.BlockSpec((TILE_R, D), lambda i, te, tv: (i, 0)),
    )
    return pl.pallas_call(
        _mlp_body,
        grid_spec=grid_spec,
        out_shape=jax.ShapeDtypeStruct((NR, D), jnp.float32),
        compiler_params=pltpu.CompilerParams(
            dimension_semantics=("arbitrary",)),
    )(texp, tvalid, xs, wi_b, bi, ws_b, bs, wo_b, bo)


def _combine_body(y_hbm, pos0_hbm, pos1_hbm, w0_hbm, w1_hbm, out_hbm,
                  idx0_v, idx1_v, w0_v, w1_v, y0_v, y1_v, sem):
    wid = lax.axis_index("s") * _NC + lax.axis_index("c")
    base = wid * _CHUNK
    pltpu.sync_copy(pos0_hbm.at[pl.ds(base, _CHUNK)], idx0_v)
    pltpu.sync_copy(pos1_hbm.at[pl.ds(base, _CHUNK)], idx1_v)
    pltpu.sync_copy(w0_hbm.at[pl.ds(base, _CHUNK)], w0_v)
    pltpu.sync_copy(w1_hbm.at[pl.ds(base, _CHUNK)], w1_v)
    pltpu.async_copy(y_hbm.at[idx0_v], y0_v, sem).wait()
    pltpu.async_copy(y_hbm.at[idx1_v], y1_v, sem).wait()

    def per_token(t, _):
        w0 = w0_v[t, :]
        w1 = w1_v[t, :]

        def per_vec(d, __):
            sl = pl.ds(d * _LANES, _LANES)
            y0_v[t, sl] = w0 * y0_v[t, sl] + w1 * y1_v[t, sl]
            return 0

        return lax.fori_loop(0, D // _LANES, per_vec, 0)

    lax.fori_loop(0, _CHUNK, per_token, 0)
    pltpu.sync_copy(y0_v, out_hbm.at[pl.ds(base, _CHUNK)])


def _run_combine(y, pos0, pos1, w):
    mesh = plsc.VectorSubcoreMesh(core_axis_name="c", subcore_axis_name="s")
    return pl.kernel(
        _combine_body,
        out_type=jax.ShapeDtypeStruct((T, D), jnp.float32),
        mesh=mesh,
        scratch_types=[
            pltpu.VMEM((_CHUNK,), jnp.int32),
            pltpu.VMEM((_CHUNK,), jnp.int32),
            pltpu.VMEM((_CHUNK, _LANES), jnp.float32),
            pltpu.VMEM((_CHUNK, _LANES), jnp.float32),
            pltpu.VMEM((_CHUNK, D), jnp.float32),
            pltpu.VMEM((_CHUNK, D), jnp.float32),
            pltpu.SemaphoreType.DMA,
        ],
    )(y, pos0, pos1,
      jnp.broadcast_to(w[:, 0:1], (T, _LANES)),
      jnp.broadcast_to(w[:, 1:2], (T, _LANES)))


def kernel(states, Wg, Wi, bi, Ws, bs, Wo, bo):
    Bv, Tv, Dv = states.shape
    x = states.reshape(Tv, Dv)

    wg_pad = jnp.zeros((D, 128), jnp.float32).at[:, :E].set(Wg)
    misc, meta = _run_router(x, wg_pad)

    w = misc[:, :2]
    pos0 = jnp.round(misc[:, 2]).astype(jnp.int32)
    pos1 = jnp.round(misc[:, 3]).astype(jnp.int32)
    texp = jnp.round(meta[:NT, 0]).astype(jnp.int32)
    tvalid = jnp.round(meta[:NT, 1]).astype(jnp.int32)

    xs = _run_dispatch(x, pos0, pos1)

    y = _run_mlp(texp, tvalid, xs,
                 Wi.astype(jnp.bfloat16), bi.reshape(E, 1, F),
                 Ws.astype(jnp.bfloat16), bs.reshape(E, 1, F),
                 Wo.astype(jnp.bfloat16), bo.reshape(E, 1, D))

    out = _run_combine(y, pos0, pos1, w)
    return out.reshape(Bv, Tv, Dv)


# speedup vs baseline: 3.1821x; 1.0686x over previous
import functools

import jax
import jax.numpy as jnp
from jax import lax
from jax.experimental import pallas as pl
from jax.experimental.pallas import tpu as pltpu
from jax.experimental.pallas import tpu_sc as plsc

D = 768
F = 3072
E = 8
T = 2048
TILE_R = 256
NT = 24
NR = NT * TILE_R

_NC, _NS = 2, 16
_NW = _NC * _NS
_CHUNK = T // _NW
_LANES = 16


def _router_body(x_ref, wg_ref, w0_ref, w1_ref, pos0_ref, pos1_ref,
                 texp_ref, tvalid_ref, strict_ref, oh_ref):
    x = x_ref[...]
    logits = jnp.dot(x, wg_ref[...], preferred_element_type=jnp.float32)

    lane = lax.broadcasted_iota(jnp.int32, (T, 128), 1)
    validl = lane < E
    l = jnp.where(validl, logits, -1e30)

    m1 = jnp.max(l, axis=1, keepdims=True)
    i1 = jnp.min(jnp.where((l == m1) & validl, lane, 999), axis=1,
                 keepdims=True)
    l2 = jnp.where(lane == i1, -1e30, l)
    m2 = jnp.max(l2, axis=1, keepdims=True)
    i2 = jnp.min(jnp.where((l2 == m2) & validl & (lane != i1), lane, 999),
                 axis=1, keepdims=True)
    s = jnp.exp(m2 - m1)
    w0 = 1.0 / (1.0 + s)
    w1 = s * w0

    oh0 = (lane == i1).astype(jnp.float32)
    oh1 = (lane == i2).astype(jnp.float32)
    oh_ref[...] = oh0 + oh1

    tri = (lax.broadcasted_iota(jnp.int32, (128, 128), 0) >
           lax.broadcasted_iota(jnp.int32, (128, 128), 1)
           ).astype(jnp.float32)

    def chunk(j, carry):
        blk = oh_ref[pl.ds(j * 128, 128), :]
        strict_ref[pl.ds(j * 128, 128), :] = (
            jnp.dot(tri, blk, preferred_element_type=jnp.float32) + carry)
        return carry + jnp.sum(blk, axis=0, keepdims=True)

    cnt = lax.fori_loop(0, T // 128, chunk,
                        jnp.zeros((1, 128), jnp.float32))

    cnt_i = jnp.round(cnt).astype(jnp.int32)
    pc_i = ((cnt_i + (TILE_R - 1)) // TILE_R) * TILE_R
    triu = (lax.broadcasted_iota(jnp.int32, (128, 128), 0) <
            lax.broadcasted_iota(jnp.int32, (128, 128), 1)
            ).astype(jnp.float32)
    offs_f = jnp.dot(pc_i.astype(jnp.float32), triu,
                     preferred_element_type=jnp.float32)
    offs_i = jnp.round(offs_f).astype(jnp.int32)

    strict = strict_ref[...]
    pos0 = jnp.sum((offs_f + strict) * oh0, axis=1, keepdims=True)
    pos1 = jnp.sum((offs_f + strict) * oh1, axis=1, keepdims=True)

    w0_ref[...] = jnp.broadcast_to(w0, (T, _LANES))
    w1_ref[...] = jnp.broadcast_to(w1, (T, _LANES))
    pos0_ref[...] = (pos0 + 0.5).astype(jnp.int32)
    pos1_ref[...] = (pos1 + 0.5).astype(jnp.int32)

    tile_end = (offs_i + pc_i) // TILE_R
    j2 = lax.broadcasted_iota(jnp.int32, (128, 128), 0)
    e2 = lax.broadcasted_iota(jnp.int32, (128, 128), 1)
    ge = ((j2 >= tile_end) & (e2 < E)).astype(jnp.int32)
    texp = jnp.minimum(jnp.sum(ge, axis=1, keepdims=True), E - 1)
    total_tiles = jnp.max(tile_end, axis=1, keepdims=True)
    jcol = lax.broadcasted_iota(jnp.int32, (128, 1), 0)
    tvalid = (jcol < total_tiles).astype(jnp.int32)
    texp_ref[...] = texp[:NT, :]
    tvalid_ref[...] = tvalid[:NT, :]


def _run_router(x, wg_pad):
    full = lambda: (0, 0)
    return pl.pallas_call(
        _router_body,
        out_shape=(jax.ShapeDtypeStruct((T, _LANES), jnp.float32),
                   jax.ShapeDtypeStruct((T, _LANES), jnp.float32),
                   jax.ShapeDtypeStruct((T, 1), jnp.int32),
                   jax.ShapeDtypeStruct((T, 1), jnp.int32),
                   jax.ShapeDtypeStruct((NT, 1), jnp.int32),
                   jax.ShapeDtypeStruct((NT, 1), jnp.int32)),
        in_specs=[pl.BlockSpec((T, D), full),
                  pl.BlockSpec((D, 128), full)],
        out_specs=(pl.BlockSpec((T, _LANES), full),
                   pl.BlockSpec((T, _LANES), full),
                   pl.BlockSpec((T, 1), full),
                   pl.BlockSpec((T, 1), full),
                   pl.BlockSpec((NT, 1), full),
                   pl.BlockSpec((NT, 1), full)),
        scratch_shapes=[pltpu.VMEM((T, 128), jnp.float32),
                        pltpu.VMEM((T, 128), jnp.float32)],
    )(x, wg_pad)


def _dispatch_body(x_hbm, pos0_hbm, pos1_hbm, xs_hbm,
                   idx0_v, idx1_v, rows_v, sem):
    wid = lax.axis_index("s") * _NC + lax.axis_index("c")
    base = wid * _CHUNK
    pltpu.sync_copy(pos0_hbm.at[pl.ds(base, _CHUNK)], idx0_v)
    pltpu.sync_copy(pos1_hbm.at[pl.ds(base, _CHUNK)], idx1_v)
    pltpu.sync_copy(x_hbm.at[pl.ds(base, _CHUNK)], rows_v)
    pltpu.async_copy(rows_v, xs_hbm.at[idx0_v], sem).wait()
    pltpu.async_copy(rows_v, xs_hbm.at[idx1_v], sem).wait()


def _run_dispatch(x, pos0, pos1):
    mesh = plsc.VectorSubcoreMesh(core_axis_name="c", subcore_axis_name="s")
    return pl.kernel(
        _dispatch_body,
        out_type=jax.ShapeDtypeStruct((NR, D), jnp.float32),
        mesh=mesh,
        scratch_types=[
            pltpu.VMEM((_CHUNK,), jnp.int32),
            pltpu.VMEM((_CHUNK,), jnp.int32),
            pltpu.VMEM((_CHUNK, D), jnp.float32),
            pltpu.SemaphoreType.DMA,
        ],
    )(x, pos0, pos1)


def _mlp_body(texp_ref, tvalid_ref, xs_ref, wi_ref, bi_ref, ws_ref, bs_ref,
              wo_ref, bo_ref, y_ref):
    i = pl.program_id(0)

    @pl.when(tvalid_ref[i] == 1)
    def _():
        xb = xs_ref[...].astype(jnp.bfloat16)
        h = jnp.dot(xb, wi_ref[0], preferred_element_type=jnp.float32)
        h = h + bi_ref[0]
        g = 0.5 * h * (1.0 + lax.erf(h * 0.7071067811865476))
        sc = jnp.dot(xb, ws_ref[0], preferred_element_type=jnp.float32)
        sc = sc + bs_ref[0]
        inner = (g * sc).astype(jnp.bfloat16)
        y = jnp.dot(inner, wo_ref[0], preferred_element_type=jnp.float32)
        y_ref[...] = y + bo_ref[0]


def _run_mlp(texp, tvalid, xs, wi_b, bi, ws_b, bs, wo_b, bo):
    grid_spec = pltpu.PrefetchScalarGridSpec(
        num_scalar_prefetch=2,
        grid=(NT,),
        in_specs=[
            pl.BlockSpec((TILE_R, D), lambda i, te, tv: (i, 0)),
            pl.BlockSpec((1, D, F), lambda i, te, tv: (te[i], 0, 0)),
            pl.BlockSpec((1, 1, F), lambda i, te, tv: (te[i], 0, 0)),
            pl.BlockSpec((1, D, F), lambda i, te, tv: (te[i], 0, 0)),
            pl.BlockSpec((1, 1, F), lambda i, te, tv: (te[i], 0, 0)),
            pl.BlockSpec((1, F, D), lambda i, te, tv: (te[i], 0, 0)),
            pl.BlockSpec((1, 1, D), lambda i, te, tv: (te[i], 0, 0)),
        ],
        out_specs=pl.BlockSpec((TILE_R, D), lambda i, te, tv: (i, 0)),
    )
    return pl.pallas_call(
        _mlp_body,
        grid_spec=grid_spec,
        out_shape=jax.ShapeDtypeStruct((NR, D), jnp.float32),
        compiler_params=pltpu.CompilerParams(
            dimension_semantics=("arbitrary",)),
    )(texp, tvalid, xs, wi_b, bi, ws_b, bs, wo_b, bo)


def _combine_body(y_hbm, pos0_hbm, pos1_hbm, w0_hbm, w1_hbm, out_hbm,
                  idx0_v, idx1_v, w0_v, w1_v, y0_v, y1_v, sem):
    wid = lax.axis_index("s") * _NC + lax.axis_index("c")
    base = wid * _CHUNK
    pltpu.sync_copy(pos0_hbm.at[pl.ds(base, _CHUNK)], idx0_v)
    pltpu.sync_copy(pos1_hbm.at[pl.ds(base, _CHUNK)], idx1_v)
    pltpu.sync_copy(w0_hbm.at[pl.ds(base, _CHUNK)], w0_v)
    pltpu.sync_copy(w1_hbm.at[pl.ds(base, _CHUNK)], w1_v)
    pltpu.async_copy(y_hbm.at[idx0_v], y0_v, sem).wait()
    pltpu.async_copy(y_hbm.at[idx1_v], y1_v, sem).wait()

    def per_token(t, _):
        w0 = w0_v[t, :]
        w1 = w1_v[t, :]
        for d in range(D // _LANES):
            sl = pl.ds(d * _LANES, _LANES)
            y0_v[t, sl] = w0 * y0_v[t, sl] + w1 * y1_v[t, sl]
        return 0

    lax.fori_loop(0, _CHUNK, per_token, 0)
    pltpu.sync_copy(y0_v, out_hbm.at[pl.ds(base, _CHUNK)])


def _run_combine(y, pos0, pos1, w0b, w1b):
    mesh = plsc.VectorSubcoreMesh(core_axis_name="c", subcore_axis_name="s")
    return pl.kernel(
        _combine_body,
        out_type=jax.ShapeDtypeStruct((T, D), jnp.float32),
        mesh=mesh,
        scratch_types=[
            pltpu.VMEM((_CHUNK,), jnp.int32),
            pltpu.VMEM((_CHUNK,), jnp.int32),
            pltpu.VMEM((_CHUNK, _LANES), jnp.float32),
            pltpu.VMEM((_CHUNK, _LANES), jnp.float32),
            pltpu.VMEM((_CHUNK, D), jnp.float32),
            pltpu.VMEM((_CHUNK, D), jnp.float32),
            pltpu.SemaphoreType.DMA,
        ],
    )(y, pos0, pos1, w0b, w1b)


def kernel(states, Wg, Wi, bi, Ws, bs, Wo, bo):
    Bv, Tv, Dv = states.shape
    x = states.reshape(Tv, Dv)

    wg_pad = jnp.zeros((D, 128), jnp.float32).at[:, :E].set(Wg)
    w0b, w1b, pos0c, pos1c, texpc, tvalidc = _run_router(x, wg_pad)

    pos0 = pos0c.reshape(T)
    pos1 = pos1c.reshape(T)
    texp = texpc.reshape(NT)
    tvalid = tvalidc.reshape(NT)

    xs = _run_dispatch(x, pos0, pos1)

    y = _run_mlp(texp, tvalid, xs,
                 Wi.astype(jnp.bfloat16), bi.reshape(E, 1, F),
                 Ws.astype(jnp.bfloat16), bs.reshape(E, 1, F),
                 Wo.astype(jnp.bfloat16), bo.reshape(E, 1, D))

    out = _run_combine(y, pos0, pos1, w0b, w1b)
    return out.reshape(Bv, Tv, Dv)


# speedup vs baseline: 3.8182x; 1.1999x over previous
import functools

import jax
import jax.numpy as jnp
from jax import lax
from jax.experimental import pallas as pl
from jax.experimental.pallas import tpu as pltpu
from jax.experimental.pallas import tpu_sc as plsc

D = 768
F = 3072
E = 8
T = 2048
TILE_R = 256
NT = 24
NR = NT * TILE_R

_NC, _NS = 2, 16
_NW = _NC * _NS
_CHUNK = T // _NW
_LANES = 16


def _router_body(x_ref, wg_ref, w0_ref, w1_ref, pos0_ref, pos1_ref,
                 texp_ref, tvalid_ref, strict_ref, oh_ref):
    x = x_ref[...]
    logits = jnp.dot(x, wg_ref[...], preferred_element_type=jnp.float32)

    lane = lax.broadcasted_iota(jnp.int32, (T, 128), 1)
    validl = lane < E
    l = jnp.where(validl, logits, -1e30)

    m1 = jnp.max(l, axis=1, keepdims=True)
    i1 = jnp.min(jnp.where((l == m1) & validl, lane, 999), axis=1,
                 keepdims=True)
    l2 = jnp.where(lane == i1, -1e30, l)
    m2 = jnp.max(l2, axis=1, keepdims=True)
    i2 = jnp.min(jnp.where((l2 == m2) & validl & (lane != i1), lane, 999),
                 axis=1, keepdims=True)
    s = jnp.exp(m2 - m1)
    w0 = 1.0 / (1.0 + s)
    w1 = s * w0

    oh0 = (lane == i1).astype(jnp.float32)
    oh1 = (lane == i2).astype(jnp.float32)
    oh_ref[...] = oh0 + oh1

    tri = (lax.broadcasted_iota(jnp.int32, (128, 128), 0) >
           lax.broadcasted_iota(jnp.int32, (128, 128), 1)
           ).astype(jnp.float32)

    def chunk(j, carry):
        blk = oh_ref[pl.ds(j * 128, 128), :]
        strict_ref[pl.ds(j * 128, 128), :] = (
            jnp.dot(tri, blk, preferred_element_type=jnp.float32) + carry)
        return carry + jnp.sum(blk, axis=0, keepdims=True)

    cnt = lax.fori_loop(0, T // 128, chunk,
                        jnp.zeros((1, 128), jnp.float32))

    cnt_i = jnp.round(cnt).astype(jnp.int32)
    pc_i = ((cnt_i + (TILE_R - 1)) // TILE_R) * TILE_R
    triu = (lax.broadcasted_iota(jnp.int32, (128, 128), 0) <
            lax.broadcasted_iota(jnp.int32, (128, 128), 1)
            ).astype(jnp.float32)
    offs_f = jnp.dot(pc_i.astype(jnp.float32), triu,
                     preferred_element_type=jnp.float32)
    offs_i = jnp.round(offs_f).astype(jnp.int32)

    strict = strict_ref[...]
    pos0 = jnp.sum((offs_f + strict) * oh0, axis=1, keepdims=True)
    pos1 = jnp.sum((offs_f + strict) * oh1, axis=1, keepdims=True)

    w0_ref[...] = jnp.broadcast_to(w0, (T, _LANES))
    w1_ref[...] = jnp.broadcast_to(w1, (T, _LANES))
    pos0_ref[...] = (pos0 + 0.5).astype(jnp.int32)
    pos1_ref[...] = (pos1 + 0.5).astype(jnp.int32)

    tile_end = (offs_i + pc_i) // TILE_R
    j2 = lax.broadcasted_iota(jnp.int32, (128, 128), 0)
    e2 = lax.broadcasted_iota(jnp.int32, (128, 128), 1)
    ge = ((j2 >= tile_end) & (e2 < E)).astype(jnp.int32)
    texp = jnp.minimum(jnp.sum(ge, axis=1, keepdims=True), E - 1)
    total_tiles = jnp.max(tile_end, axis=1, keepdims=True)
    jcol = lax.broadcasted_iota(jnp.int32, (128, 1), 0)
    tvalid = (jcol < total_tiles).astype(jnp.int32)
    texp_ref[...] = texp[:NT, :]
    tvalid_ref[...] = tvalid[:NT, :]


def _run_router(x, wg_pad):
    full = lambda: (0, 0)
    return pl.pallas_call(
        _router_body,
        out_shape=(jax.ShapeDtypeStruct((T, _LANES), jnp.float32),
                   jax.ShapeDtypeStruct((T, _LANES), jnp.float32),
                   jax.ShapeDtypeStruct((T, 1), jnp.int32),
                   jax.ShapeDtypeStruct((T, 1), jnp.int32),
                   jax.ShapeDtypeStruct((NT, 1), jnp.int32),
                   jax.ShapeDtypeStruct((NT, 1), jnp.int32)),
        in_specs=[pl.BlockSpec((T, D), full),
                  pl.BlockSpec((D, 128), full)],
        out_specs=(pl.BlockSpec((T, _LANES), full),
                   pl.BlockSpec((T, _LANES), full),
                   pl.BlockSpec((T, 1), full),
                   pl.BlockSpec((T, 1), full),
                   pl.BlockSpec((NT, 1), full),
                   pl.BlockSpec((NT, 1), full)),
        scratch_shapes=[pltpu.VMEM((T, 128), jnp.float32),
                        pltpu.VMEM((T, 128), jnp.float32)],
    )(x, wg_pad)


def _dispatch_body(x_hbm, pos0_hbm, pos1_hbm, xs_hbm,
                   idx0_v, idx1_v, rows_v, sem):
    wid = lax.axis_index("s") * _NC + lax.axis_index("c")
    base = wid * _CHUNK
    pltpu.sync_copy(pos0_hbm.at[pl.ds(base, _CHUNK)], idx0_v)
    pltpu.sync_copy(pos1_hbm.at[pl.ds(base, _CHUNK)], idx1_v)
    pltpu.sync_copy(x_hbm.at[pl.ds(base, _CHUNK)], rows_v)
    pltpu.async_copy(rows_v, xs_hbm.at[idx0_v], sem).wait()
    pltpu.async_copy(rows_v, xs_hbm.at[idx1_v], sem).wait()


def _run_dispatch(x, pos0, pos1):
    mesh = plsc.VectorSubcoreMesh(core_axis_name="c", subcore_axis_name="s")
    return pl.kernel(
        _dispatch_body,
        out_type=jax.ShapeDtypeStruct((NR, D), jnp.float32),
        mesh=mesh,
        scratch_types=[
            pltpu.VMEM((_CHUNK,), jnp.int32),
            pltpu.VMEM((_CHUNK,), jnp.int32),
            pltpu.VMEM((_CHUNK, D), jnp.float32),
            pltpu.SemaphoreType.DMA,
        ],
    )(x, pos0, pos1)


FH = F // 2


def _mlp_half(xb, wi_ref, bi_ref, ws_ref, bs_ref, wo_ref):
    h = jnp.dot(xb, wi_ref[0].astype(jnp.bfloat16),
                preferred_element_type=jnp.float32)
    h = h + bi_ref[0]
    g = 0.5 * h * (1.0 + lax.erf(h * 0.7071067811865476))
    sc = jnp.dot(xb, ws_ref[0].astype(jnp.bfloat16),
                 preferred_element_type=jnp.float32)
    sc = sc + bs_ref[0]
    inner = (g * sc).astype(jnp.bfloat16)
    return jnp.dot(inner, wo_ref[0].astype(jnp.bfloat16),
                   preferred_element_type=jnp.float32)


def _mlp_body_a(texp_ref, tvalid_ref, xs_ref, wi_ref, bi_ref,
                ws_ref, bs_ref, wo_ref, bo_ref, y_ref):
    i = pl.program_id(0)

    @pl.when(tvalid_ref[i] == 1)
    def _():
        xb = xs_ref[...].astype(jnp.bfloat16)
        y_ref[...] = _mlp_half(xb, wi_ref, bi_ref, ws_ref, bs_ref,
                               wo_ref) + bo_ref[0]


def _mlp_body_b(texp_ref, tvalid_ref, xs_ref, yin_ref, wi_ref, bi_ref,
                ws_ref, bs_ref, wo_ref, y_ref):
    i = pl.program_id(0)

    @pl.when(tvalid_ref[i] == 1)
    def _():
        xb = xs_ref[...].astype(jnp.bfloat16)
        y_ref[...] = yin_ref[...] + _mlp_half(xb, wi_ref, bi_ref, ws_ref,
                                              bs_ref, wo_ref)


def _run_mlp(texp, tvalid, xs, wi, bi, ws, bs, wo, bo):
    row = lambda i, te, tv: (i, 0)

    def wspecs(nf):
        return [
            pl.BlockSpec((1, D, FH), lambda i, te, tv: (te[i], 0, nf)),
            pl.BlockSpec((1, 1, FH), lambda i, te, tv: (te[i], 0, nf)),
            pl.BlockSpec((1, D, FH), lambda i, te, tv: (te[i], 0, nf)),
            pl.BlockSpec((1, 1, FH), lambda i, te, tv: (te[i], 0, nf)),
            pl.BlockSpec((1, FH, D), lambda i, te, tv: (te[i], nf, 0)),
        ]

    grid_a = pltpu.PrefetchScalarGridSpec(
        num_scalar_prefetch=2, grid=(NT,),
        in_specs=[pl.BlockSpec((TILE_R, D), row)] + wspecs(0) + [
            pl.BlockSpec((1, 1, D), lambda i, te, tv: (te[i], 0, 0))],
        out_specs=pl.BlockSpec((TILE_R, D), row),
    )
    y0 = pl.pallas_call(
        _mlp_body_a,
        grid_spec=grid_a,
        out_shape=jax.ShapeDtypeStruct((NR, D), jnp.float32),
        compiler_params=pltpu.CompilerParams(
            dimension_semantics=("arbitrary",)),
    )(texp, tvalid, xs, wi, bi, ws, bs, wo, bo)

    grid_b = pltpu.PrefetchScalarGridSpec(
        num_scalar_prefetch=2, grid=(NT,),
        in_specs=[pl.BlockSpec((TILE_R, D), row),
                  pl.BlockSpec((TILE_R, D), row)] + wspecs(1),
        out_specs=pl.BlockSpec((TILE_R, D), row),
    )
    return pl.pallas_call(
        _mlp_body_b,
        grid_spec=grid_b,
        out_shape=jax.ShapeDtypeStruct((NR, D), jnp.float32),
        input_output_aliases={3: 0},
        compiler_params=pltpu.CompilerParams(
            dimension_semantics=("arbitrary",)),
    )(texp, tvalid, xs, y0, wi, bi, ws, bs, wo)


def _combine_body(y_hbm, pos0_hbm, pos1_hbm, w0_hbm, w1_hbm, out_hbm,
                  idx0_v, idx1_v, w0_v, w1_v, y0_v, y1_v, sem):
    wid = lax.axis_index("s") * _NC + lax.axis_index("c")
    base = wid * _CHUNK
    pltpu.sync_copy(pos0_hbm.at[pl.ds(base, _CHUNK)], idx0_v)
    pltpu.sync_copy(pos1_hbm.at[pl.ds(base, _CHUNK)], idx1_v)
    pltpu.sync_copy(w0_hbm.at[pl.ds(base, _CHUNK)], w0_v)
    pltpu.sync_copy(w1_hbm.at[pl.ds(base, _CHUNK)], w1_v)
    pltpu.async_copy(y_hbm.at[idx0_v], y0_v, sem).wait()
    pltpu.async_copy(y_hbm.at[idx1_v], y1_v, sem).wait()

    def per_token(t, _):
        w0 = w0_v[t, :]
        w1 = w1_v[t, :]
        for d in range(D // _LANES):
            sl = pl.ds(d * _LANES, _LANES)
            y0_v[t, sl] = w0 * y0_v[t, sl] + w1 * y1_v[t, sl]
        return 0

    lax.fori_loop(0, _CHUNK, per_token, 0)
    pltpu.sync_copy(y0_v, out_hbm.at[pl.ds(base, _CHUNK)])


def _run_combine(y, pos0, pos1, w0b, w1b):
    mesh = plsc.VectorSubcoreMesh(core_axis_name="c", subcore_axis_name="s")
    return pl.kernel(
        _combine_body,
        out_type=jax.ShapeDtypeStruct((T, D), jnp.float32),
        mesh=mesh,
        scratch_types=[
            pltpu.VMEM((_CHUNK,), jnp.int32),
            pltpu.VMEM((_CHUNK,), jnp.int32),
            pltpu.VMEM((_CHUNK, _LANES), jnp.float32),
            pltpu.VMEM((_CHUNK, _LANES), jnp.float32),
            pltpu.VMEM((_CHUNK, D), jnp.float32),
            pltpu.VMEM((_CHUNK, D), jnp.float32),
            pltpu.SemaphoreType.DMA,
        ],
    )(y, pos0, pos1, w0b, w1b)


def kernel(states, Wg, Wi, bi, Ws, bs, Wo, bo):
    Bv, Tv, Dv = states.shape
    x = states.reshape(Tv, Dv)

    wg_pad = jnp.zeros((D, 128), jnp.float32).at[:, :E].set(Wg)
    w0b, w1b, pos0c, pos1c, texpc, tvalidc = _run_router(x, wg_pad)

    pos0 = pos0c.reshape(T)
    pos1 = pos1c.reshape(T)
    texp = texpc.reshape(NT)
    tvalid = tvalidc.reshape(NT)

    xs = _run_dispatch(x, pos0, pos1)

    y = _run_mlp(texp, tvalid, xs,
                 Wi, bi.reshape(E, 1, F),
                 Ws, bs.reshape(E, 1, F),
                 Wo, bo.reshape(E, 1, D))

    out = _run_combine(y, pos0, pos1, w0b, w1b)
    return out.reshape(Bv, Tv, Dv)


# speedup vs baseline: 3.8548x; 1.0096x over previous
import functools

import jax
import jax.numpy as jnp
from jax import lax
from jax.experimental import pallas as pl
from jax.experimental.pallas import tpu as pltpu
from jax.experimental.pallas import tpu_sc as plsc

D = 768
F = 3072
E = 8
T = 2048
TILE_R = 256
NT = 24
NR = NT * TILE_R

_NC, _NS = 2, 16
_NW = _NC * _NS
_CHUNK = T // _NW
_LANES = 16


def _router_body(x_ref, wg_ref, w0_ref, w1_ref, pos0_ref, pos1_ref,
                 texp_ref, tvalid_ref, strict_ref, oh_ref):
    x = x_ref[...]
    logits = jnp.dot(x, wg_ref[...], preferred_element_type=jnp.float32)

    lane = lax.broadcasted_iota(jnp.int32, (T, 128), 1)
    validl = lane < E
    l = jnp.where(validl, logits, -1e30)

    m1 = jnp.max(l, axis=1, keepdims=True)
    i1 = jnp.min(jnp.where((l == m1) & validl, lane, 999), axis=1,
                 keepdims=True)
    l2 = jnp.where(lane == i1, -1e30, l)
    m2 = jnp.max(l2, axis=1, keepdims=True)
    i2 = jnp.min(jnp.where((l2 == m2) & validl & (lane != i1), lane, 999),
                 axis=1, keepdims=True)
    s = jnp.exp(m2 - m1)
    w0 = 1.0 / (1.0 + s)
    w1 = s * w0

    oh0 = (lane == i1).astype(jnp.float32)
    oh1 = (lane == i2).astype(jnp.float32)
    oh_ref[...] = oh0 + oh1

    tri = (lax.broadcasted_iota(jnp.int32, (128, 128), 0) >
           lax.broadcasted_iota(jnp.int32, (128, 128), 1)
           ).astype(jnp.float32)

    def chunk(j, carry):
        blk = oh_ref[pl.ds(j * 128, 128), :]
        strict_ref[pl.ds(j * 128, 128), :] = (
            jnp.dot(tri, blk, preferred_element_type=jnp.float32) + carry)
        return carry + jnp.sum(blk, axis=0, keepdims=True)

    cnt = lax.fori_loop(0, T // 128, chunk,
                        jnp.zeros((1, 128), jnp.float32))

    cnt_i = jnp.round(cnt).astype(jnp.int32)
    pc_i = ((cnt_i + (TILE_R - 1)) // TILE_R) * TILE_R
    triu = (lax.broadcasted_iota(jnp.int32, (128, 128), 0) <
            lax.broadcasted_iota(jnp.int32, (128, 128), 1)
            ).astype(jnp.float32)
    offs_f = jnp.dot(pc_i.astype(jnp.float32), triu,
                     preferred_element_type=jnp.float32)
    offs_i = jnp.round(offs_f).astype(jnp.int32)

    strict = strict_ref[...]
    pos0 = jnp.sum((offs_f + strict) * oh0, axis=1, keepdims=True)
    pos1 = jnp.sum((offs_f + strict) * oh1, axis=1, keepdims=True)

    w0_ref[...] = jnp.broadcast_to(w0, (T, _LANES))
    w1_ref[...] = jnp.broadcast_to(w1, (T, _LANES))
    pos0_ref[...] = (pos0 + 0.5).astype(jnp.int32)
    pos1_ref[...] = (pos1 + 0.5).astype(jnp.int32)

    tile_end = (offs_i + pc_i) // TILE_R
    j2 = lax.broadcasted_iota(jnp.int32, (128, 128), 0)
    e2 = lax.broadcasted_iota(jnp.int32, (128, 128), 1)
    ge = ((j2 >= tile_end) & (e2 < E)).astype(jnp.int32)
    texp = jnp.minimum(jnp.sum(ge, axis=1, keepdims=True), E - 1)
    total_tiles = jnp.max(tile_end, axis=1, keepdims=True)
    jcol = lax.broadcasted_iota(jnp.int32, (128, 1), 0)
    tvalid = (jcol < total_tiles).astype(jnp.int32)
    texp_ref[...] = texp[:NT, :]
    tvalid_ref[...] = tvalid[:NT, :]


def _run_router(x, wg_pad):
    full = lambda: (0, 0)
    return pl.pallas_call(
        _router_body,
        out_shape=(jax.ShapeDtypeStruct((T, _LANES), jnp.float32),
                   jax.ShapeDtypeStruct((T, _LANES), jnp.float32),
                   jax.ShapeDtypeStruct((T, 1), jnp.int32),
                   jax.ShapeDtypeStruct((T, 1), jnp.int32),
                   jax.ShapeDtypeStruct((NT, 1), jnp.int32),
                   jax.ShapeDtypeStruct((NT, 1), jnp.int32)),
        in_specs=[pl.BlockSpec((T, D), full),
                  pl.BlockSpec((D, 128), full)],
        out_specs=(pl.BlockSpec((T, _LANES), full),
                   pl.BlockSpec((T, _LANES), full),
                   pl.BlockSpec((T, 1), full),
                   pl.BlockSpec((T, 1), full),
                   pl.BlockSpec((NT, 1), full),
                   pl.BlockSpec((NT, 1), full)),
        scratch_shapes=[pltpu.VMEM((T, 128), jnp.float32),
                        pltpu.VMEM((T, 128), jnp.float32)],
    )(x, wg_pad)


def _dispatch_body(x_hbm, pos0_hbm, pos1_hbm, xs_hbm,
                   idx0_v, idx1_v, rows_v, sem):
    wid = lax.axis_index("s") * _NC + lax.axis_index("c")
    base = wid * _CHUNK
    pltpu.sync_copy(pos0_hbm.at[pl.ds(base, _CHUNK)], idx0_v)
    pltpu.sync_copy(pos1_hbm.at[pl.ds(base, _CHUNK)], idx1_v)
    pltpu.sync_copy(x_hbm.at[pl.ds(base, _CHUNK)], rows_v)
    pltpu.async_copy(rows_v, xs_hbm.at[idx0_v], sem).wait()
    pltpu.async_copy(rows_v, xs_hbm.at[idx1_v], sem).wait()


def _run_dispatch(x, pos0, pos1):
    mesh = plsc.VectorSubcoreMesh(core_axis_name="c", subcore_axis_name="s")
    return pl.kernel(
        _dispatch_body,
        out_type=jax.ShapeDtypeStruct((NR, D), jnp.float32),
        mesh=mesh,
        scratch_types=[
            pltpu.VMEM((_CHUNK,), jnp.int32),
            pltpu.VMEM((_CHUNK,), jnp.int32),
            pltpu.VMEM((_CHUNK, D), jnp.float32),
            pltpu.SemaphoreType.DMA,
        ],
    )(x, pos0, pos1)


FH = F // 2


def _mlp_half(xb, wi_ref, bi_ref, ws_ref, bs_ref, wo_ref):
    h = jnp.dot(xb, wi_ref[0].astype(jnp.bfloat16),
                preferred_element_type=jnp.float32)
    h = h + bi_ref[0]
    g = 0.5 * h * (1.0 + lax.erf(h * 0.7071067811865476))
    sc = jnp.dot(xb, ws_ref[0].astype(jnp.bfloat16),
                 preferred_element_type=jnp.float32)
    sc = sc + bs_ref[0]
    inner = (g * sc).astype(jnp.bfloat16)
    return jnp.dot(inner, wo_ref[0].astype(jnp.bfloat16),
                   preferred_element_type=jnp.float32)


def _mlp_body_a(texp_ref, tvalid_ref, xs_ref, wi_ref, bi_ref,
                ws_ref, bs_ref, wo_ref, bo_ref, y_ref):
    i = pl.program_id(0)

    @pl.when(tvalid_ref[i] == 1)
    def _():
        xb = xs_ref[...].astype(jnp.bfloat16)
        y_ref[...] = (_mlp_half(xb, wi_ref, bi_ref, ws_ref, bs_ref,
                                wo_ref) + bo_ref[0]).astype(jnp.bfloat16)


def _mlp_body_b(texp_ref, tvalid_ref, xs_ref, yin_ref, wi_ref, bi_ref,
                ws_ref, bs_ref, wo_ref, y_ref):
    i = pl.program_id(0)

    @pl.when(tvalid_ref[i] == 1)
    def _():
        xb = xs_ref[...].astype(jnp.bfloat16)
        y_ref[...] = (yin_ref[...].astype(jnp.float32) +
                      _mlp_half(xb, wi_ref, bi_ref, ws_ref, bs_ref, wo_ref))


def _run_mlp(texp, tvalid, xs, wi, bi, ws, bs, wo, bo):
    row = lambda i, te, tv: (i, 0)

    def wspecs(nf):
        return [
            pl.BlockSpec((1, D, FH), lambda i, te, tv: (te[i], 0, nf)),
            pl.BlockSpec((1, 1, FH), lambda i, te, tv: (te[i], 0, nf)),
            pl.BlockSpec((1, D, FH), lambda i, te, tv: (te[i], 0, nf)),
            pl.BlockSpec((1, 1, FH), lambda i, te, tv: (te[i], 0, nf)),
            pl.BlockSpec((1, FH, D), lambda i, te, tv: (te[i], nf, 0)),
        ]

    grid_a = pltpu.PrefetchScalarGridSpec(
        num_scalar_prefetch=2, grid=(NT,),
        in_specs=[pl.BlockSpec((TILE_R, D), row)] + wspecs(0) + [
            pl.BlockSpec((1, 1, D), lambda i, te, tv: (te[i], 0, 0))],
        out_specs=pl.BlockSpec((TILE_R, D), row),
    )
    y0 = pl.pallas_call(
        _mlp_body_a,
        grid_spec=grid_a,
        out_shape=jax.ShapeDtypeStruct((NR, D), jnp.bfloat16),
        compiler_params=pltpu.CompilerParams(
            dimension_semantics=("arbitrary",)),
    )(texp, tvalid, xs, wi, bi, ws, bs, wo, bo)

    grid_b = pltpu.PrefetchScalarGridSpec(
        num_scalar_prefetch=2, grid=(NT,),
        in_specs=[pl.BlockSpec((TILE_R, D), row),
                  pl.BlockSpec((TILE_R, D), row)] + wspecs(1),
        out_specs=pl.BlockSpec((TILE_R, D), row),
    )
    return pl.pallas_call(
        _mlp_body_b,
        grid_spec=grid_b,
        out_shape=jax.ShapeDtypeStruct((NR, D), jnp.float32),
        input_output_aliases={},
        compiler_params=pltpu.CompilerParams(
            dimension_semantics=("arbitrary",)),
    )(texp, tvalid, xs, y0, wi, bi, ws, bs, wo)


def _combine_body(y_hbm, pos0_hbm, pos1_hbm, w0_hbm, w1_hbm, out_hbm,
                  idx0_v, idx1_v, w0_v, w1_v, y0_v, y1_v, sem):
    wid = lax.axis_index("s") * _NC + lax.axis_index("c")
    base = wid * _CHUNK
    pltpu.sync_copy(pos0_hbm.at[pl.ds(base, _CHUNK)], idx0_v)
    pltpu.sync_copy(pos1_hbm.at[pl.ds(base, _CHUNK)], idx1_v)
    pltpu.sync_copy(w0_hbm.at[pl.ds(base, _CHUNK)], w0_v)
    pltpu.sync_copy(w1_hbm.at[pl.ds(base, _CHUNK)], w1_v)
    pltpu.async_copy(y_hbm.at[idx0_v], y0_v, sem).wait()
    pltpu.async_copy(y_hbm.at[idx1_v], y1_v, sem).wait()

    def per_token(t, _):
        w0 = w0_v[t, :]
        w1 = w1_v[t, :]
        for d in range(D // _LANES):
            sl = pl.ds(d * _LANES, _LANES)
            y0_v[t, sl] = w0 * y0_v[t, sl] + w1 * y1_v[t, sl]
        return 0

    lax.fori_loop(0, _CHUNK, per_token, 0)
    pltpu.sync_copy(y0_v, out_hbm.at[pl.ds(base, _CHUNK)])


def _run_combine(y, pos0, pos1, w0b, w1b):
    mesh = plsc.VectorSubcoreMesh(core_axis_name="c", subcore_axis_name="s")
    return pl.kernel(
        _combine_body,
        out_type=jax.ShapeDtypeStruct((T, D), jnp.float32),
        mesh=mesh,
        scratch_types=[
            pltpu.VMEM((_CHUNK,), jnp.int32),
            pltpu.VMEM((_CHUNK,), jnp.int32),
            pltpu.VMEM((_CHUNK, _LANES), jnp.float32),
            pltpu.VMEM((_CHUNK, _LANES), jnp.float32),
            pltpu.VMEM((_CHUNK, D), jnp.float32),
            pltpu.VMEM((_CHUNK, D), jnp.float32),
            pltpu.SemaphoreType.DMA,
        ],
    )(y, pos0, pos1, w0b, w1b)


def kernel(states, Wg, Wi, bi, Ws, bs, Wo, bo):
    Bv, Tv, Dv = states.shape
    x = states.reshape(Tv, Dv)

    wg_pad = jnp.zeros((D, 128), jnp.float32).at[:, :E].set(Wg)
    w0b, w1b, pos0c, pos1c, texpc, tvalidc = _run_router(x, wg_pad)

    pos0 = pos0c.reshape(T)
    pos1 = pos1c.reshape(T)
    texp = texpc.reshape(NT)
    tvalid = tvalidc.reshape(NT)

    xs = _run_dispatch(x, pos0, pos1)

    y = _run_mlp(texp, tvalid, xs,
                 Wi, bi.reshape(E, 1, F),
                 Ws, bs.reshape(E, 1, F),
                 Wo, bo.reshape(E, 1, D))

    out = _run_combine(y, pos0, pos1, w0b, w1b)
    return out.reshape(Bv, Tv, Dv)


# speedup vs baseline: 4.2092x; 1.0919x over previous
import functools

import jax
import jax.numpy as jnp
from jax import lax
from jax.experimental import pallas as pl
from jax.experimental.pallas import tpu as pltpu
from jax.experimental.pallas import tpu_sc as plsc

D = 768
F = 3072
E = 8
T = 2048
TILE_R = 512
NT = 16
NR = NT * TILE_R

_NC, _NS = 2, 16
_NW = _NC * _NS
_CHUNK = T // _NW
_LANES = 16


def _router_body(x_ref, wg_ref, w0_ref, w1_ref, pos0_ref, pos1_ref,
                 texp_ref, tvalid_ref, strict_ref, oh_ref):
    x = x_ref[...]
    logits = jnp.dot(x, wg_ref[...], preferred_element_type=jnp.float32)

    lane = lax.broadcasted_iota(jnp.int32, (T, 128), 1)
    validl = lane < E
    l = jnp.where(validl, logits, -1e30)

    m1 = jnp.max(l, axis=1, keepdims=True)
    i1 = jnp.min(jnp.where((l == m1) & validl, lane, 999), axis=1,
                 keepdims=True)
    l2 = jnp.where(lane == i1, -1e30, l)
    m2 = jnp.max(l2, axis=1, keepdims=True)
    i2 = jnp.min(jnp.where((l2 == m2) & validl & (lane != i1), lane, 999),
                 axis=1, keepdims=True)
    s = jnp.exp(m2 - m1)
    w0 = 1.0 / (1.0 + s)
    w1 = s * w0

    oh0 = (lane == i1).astype(jnp.float32)
    oh1 = (lane == i2).astype(jnp.float32)
    oh_ref[...] = oh0 + oh1

    tri = (lax.broadcasted_iota(jnp.int32, (128, 128), 0) >
           lax.broadcasted_iota(jnp.int32, (128, 128), 1)
           ).astype(jnp.float32)

    def chunk(j, carry):
        blk = oh_ref[pl.ds(j * 128, 128), :]
        strict_ref[pl.ds(j * 128, 128), :] = (
            jnp.dot(tri, blk, preferred_element_type=jnp.float32) + carry)
        return carry + jnp.sum(blk, axis=0, keepdims=True)

    cnt = lax.fori_loop(0, T // 128, chunk,
                        jnp.zeros((1, 128), jnp.float32))

    cnt_i = jnp.round(cnt).astype(jnp.int32)
    pc_i = ((cnt_i + (TILE_R - 1)) // TILE_R) * TILE_R
    triu = (lax.broadcasted_iota(jnp.int32, (128, 128), 0) <
            lax.broadcasted_iota(jnp.int32, (128, 128), 1)
            ).astype(jnp.float32)
    offs_f = jnp.dot(pc_i.astype(jnp.float32), triu,
                     preferred_element_type=jnp.float32)
    offs_i = jnp.round(offs_f).astype(jnp.int32)

    strict = strict_ref[...]
    pos0 = jnp.sum((offs_f + strict) * oh0, axis=1, keepdims=True)
    pos1 = jnp.sum((offs_f + strict) * oh1, axis=1, keepdims=True)

    w0_ref[...] = jnp.broadcast_to(w0, (T, _LANES))
    w1_ref[...] = jnp.broadcast_to(w1, (T, _LANES))
    pos0_ref[...] = (pos0 + 0.5).astype(jnp.int32)
    pos1_ref[...] = (pos1 + 0.5).astype(jnp.int32)

    tile_end = (offs_i + pc_i) // TILE_R
    j2 = lax.broadcasted_iota(jnp.int32, (128, 128), 0)
    e2 = lax.broadcasted_iota(jnp.int32, (128, 128), 1)
    ge = ((j2 >= tile_end) & (e2 < E)).astype(jnp.int32)
    texp = jnp.minimum(jnp.sum(ge, axis=1, keepdims=True), E - 1)
    total_tiles = jnp.max(tile_end, axis=1, keepdims=True)
    jcol = lax.broadcasted_iota(jnp.int32, (128, 1), 0)
    tvalid = (jcol < total_tiles).astype(jnp.int32)
    texp_ref[...] = texp[:NT, :]
    tvalid_ref[...] = tvalid[:NT, :]


def _run_router(x, wg_pad):
    full = lambda: (0, 0)
    return pl.pallas_call(
        _router_body,
        out_shape=(jax.ShapeDtypeStruct((T, _LANES), jnp.float32),
                   jax.ShapeDtypeStruct((T, _LANES), jnp.float32),
                   jax.ShapeDtypeStruct((T, 1), jnp.int32),
                   jax.ShapeDtypeStruct((T, 1), jnp.int32),
                   jax.ShapeDtypeStruct((NT, 1), jnp.int32),
                   jax.ShapeDtypeStruct((NT, 1), jnp.int32)),
        in_specs=[pl.BlockSpec((T, D), full),
                  pl.BlockSpec((D, 128), full)],
        out_specs=(pl.BlockSpec((T, _LANES), full),
                   pl.BlockSpec((T, _LANES), full),
                   pl.BlockSpec((T, 1), full),
                   pl.BlockSpec((T, 1), full),
                   pl.BlockSpec((NT, 1), full),
                   pl.BlockSpec((NT, 1), full)),
        scratch_shapes=[pltpu.VMEM((T, 128), jnp.float32),
                        pltpu.VMEM((T, 128), jnp.float32)],
    )(x, wg_pad)


def _dispatch_body(x_hbm, pos0_hbm, pos1_hbm, xs_hbm,
                   idx0_v, idx1_v, rows_v, sem):
    wid = lax.axis_index("s") * _NC + lax.axis_index("c")
    base = wid * _CHUNK
    pltpu.sync_copy(pos0_hbm.at[pl.ds(base, _CHUNK)], idx0_v)
    pltpu.sync_copy(pos1_hbm.at[pl.ds(base, _CHUNK)], idx1_v)
    pltpu.sync_copy(x_hbm.at[pl.ds(base, _CHUNK)], rows_v)
    pltpu.async_copy(rows_v, xs_hbm.at[idx0_v], sem).wait()
    pltpu.async_copy(rows_v, xs_hbm.at[idx1_v], sem).wait()


def _run_dispatch(x, pos0, pos1):
    mesh = plsc.VectorSubcoreMesh(core_axis_name="c", subcore_axis_name="s")
    return pl.kernel(
        _dispatch_body,
        out_type=jax.ShapeDtypeStruct((NR, D), jnp.float32),
        mesh=mesh,
        scratch_types=[
            pltpu.VMEM((_CHUNK,), jnp.int32),
            pltpu.VMEM((_CHUNK,), jnp.int32),
            pltpu.VMEM((_CHUNK, D), jnp.float32),
            pltpu.SemaphoreType.DMA,
        ],
    )(x, pos0, pos1)


FH = F // 2


def _mlp_half(xb, wi_ref, bi_ref, ws_ref, bs_ref, wo_ref):
    h = jnp.dot(xb, wi_ref[0].astype(jnp.bfloat16),
                preferred_element_type=jnp.float32)
    h = h + bi_ref[0]
    g = 0.5 * h * (1.0 + lax.erf(h * 0.7071067811865476))
    sc = jnp.dot(xb, ws_ref[0].astype(jnp.bfloat16),
                 preferred_element_type=jnp.float32)
    sc = sc + bs_ref[0]
    inner = (g * sc).astype(jnp.bfloat16)
    return jnp.dot(inner, wo_ref[0].astype(jnp.bfloat16),
                   preferred_element_type=jnp.float32)


def _mlp_body_a(texp_ref, tvalid_ref, xs_ref, wi_ref, bi_ref,
                ws_ref, bs_ref, wo_ref, bo_ref, y_ref):
    i = pl.program_id(0)

    @pl.when(tvalid_ref[i] == 1)
    def _():
        xb = xs_ref[...].astype(jnp.bfloat16)
        y_ref[...] = (_mlp_half(xb, wi_ref, bi_ref, ws_ref, bs_ref,
                                wo_ref) + bo_ref[0]).astype(jnp.bfloat16)


def _mlp_body_b(texp_ref, tvalid_ref, xs_ref, yin_ref, wi_ref, bi_ref,
                ws_ref, bs_ref, wo_ref, y_ref):
    i = pl.program_id(0)

    @pl.when(tvalid_ref[i] == 1)
    def _():
        xb = xs_ref[...].astype(jnp.bfloat16)
        y_ref[...] = (yin_ref[...].astype(jnp.float32) +
                      _mlp_half(xb, wi_ref, bi_ref, ws_ref, bs_ref, wo_ref))


def _run_mlp(texp, tvalid, xs, wi, bi, ws, bs, wo, bo):
    row = lambda i, te, tv: (i, 0)

    def wspecs(nf):
        return [
            pl.BlockSpec((1, D, FH), lambda i, te, tv: (te[i], 0, nf)),
            pl.BlockSpec((1, 1, FH), lambda i, te, tv: (te[i], 0, nf)),
            pl.BlockSpec((1, D, FH), lambda i, te, tv: (te[i], 0, nf)),
            pl.BlockSpec((1, 1, FH), lambda i, te, tv: (te[i], 0, nf)),
            pl.BlockSpec((1, FH, D), lambda i, te, tv: (te[i], nf, 0)),
        ]

    grid_a = pltpu.PrefetchScalarGridSpec(
        num_scalar_prefetch=2, grid=(NT,),
        in_specs=[pl.BlockSpec((TILE_R, D), row)] + wspecs(0) + [
            pl.BlockSpec((1, 1, D), lambda i, te, tv: (te[i], 0, 0))],
        out_specs=pl.BlockSpec((TILE_R, D), row),
    )
    y0 = pl.pallas_call(
        _mlp_body_a,
        grid_spec=grid_a,
        out_shape=jax.ShapeDtypeStruct((NR, D), jnp.bfloat16),
        compiler_params=pltpu.CompilerParams(
            dimension_semantics=("arbitrary",)),
    )(texp, tvalid, xs, wi, bi, ws, bs, wo, bo)

    grid_b = pltpu.PrefetchScalarGridSpec(
        num_scalar_prefetch=2, grid=(NT,),
        in_specs=[pl.BlockSpec((TILE_R, D), row),
                  pl.BlockSpec((TILE_R, D), row)] + wspecs(1),
        out_specs=pl.BlockSpec((TILE_R, D), row),
    )
    return pl.pallas_call(
        _mlp_body_b,
        grid_spec=grid_b,
        out_shape=jax.ShapeDtypeStruct((NR, D), jnp.float32),
        input_output_aliases={},
        compiler_params=pltpu.CompilerParams(
            dimension_semantics=("arbitrary",)),
    )(texp, tvalid, xs, y0, wi, bi, ws, bs, wo)


def _combine_body(y_hbm, pos0_hbm, pos1_hbm, w0_hbm, w1_hbm, out_hbm,
                  idx0_v, idx1_v, w0_v, w1_v, y0_v, y1_v, sem):
    wid = lax.axis_index("s") * _NC + lax.axis_index("c")
    base = wid * _CHUNK
    pltpu.sync_copy(pos0_hbm.at[pl.ds(base, _CHUNK)], idx0_v)
    pltpu.sync_copy(pos1_hbm.at[pl.ds(base, _CHUNK)], idx1_v)
    pltpu.sync_copy(w0_hbm.at[pl.ds(base, _CHUNK)], w0_v)
    pltpu.sync_copy(w1_hbm.at[pl.ds(base, _CHUNK)], w1_v)
    pltpu.async_copy(y_hbm.at[idx0_v], y0_v, sem).wait()
    pltpu.async_copy(y_hbm.at[idx1_v], y1_v, sem).wait()

    def per_token(t, _):
        w0 = w0_v[t, :]
        w1 = w1_v[t, :]
        for d in range(D // _LANES):
            sl = pl.ds(d * _LANES, _LANES)
            y0_v[t, sl] = w0 * y0_v[t, sl] + w1 * y1_v[t, sl]
        return 0

    lax.fori_loop(0, _CHUNK, per_token, 0)
    pltpu.sync_copy(y0_v, out_hbm.at[pl.ds(base, _CHUNK)])


def _run_combine(y, pos0, pos1, w0b, w1b):
    mesh = plsc.VectorSubcoreMesh(core_axis_name="c", subcore_axis_name="s")
    return pl.kernel(
        _combine_body,
        out_type=jax.ShapeDtypeStruct((T, D), jnp.float32),
        mesh=mesh,
        scratch_types=[
            pltpu.VMEM((_CHUNK,), jnp.int32),
            pltpu.VMEM((_CHUNK,), jnp.int32),
            pltpu.VMEM((_CHUNK, _LANES), jnp.float32),
            pltpu.VMEM((_CHUNK, _LANES), jnp.float32),
            pltpu.VMEM((_CHUNK, D), jnp.float32),
            pltpu.VMEM((_CHUNK, D), jnp.float32),
            pltpu.SemaphoreType.DMA,
        ],
    )(y, pos0, pos1, w0b, w1b)


def kernel(states, Wg, Wi, bi, Ws, bs, Wo, bo):
    Bv, Tv, Dv = states.shape
    x = states.reshape(Tv, Dv)

    wg_pad = jnp.zeros((D, 128), jnp.float32).at[:, :E].set(Wg)
    w0b, w1b, pos0c, pos1c, texpc, tvalidc = _run_router(x, wg_pad)

    pos0 = pos0c.reshape(T)
    pos1 = pos1c.reshape(T)
    texp = texpc.reshape(NT)
    tvalid = tvalidc.reshape(NT)

    xs = _run_dispatch(x, pos0, pos1)

    y = _run_mlp(texp, tvalid, xs,
                 Wi, bi.reshape(E, 1, F),
                 Ws, bs.reshape(E, 1, F),
                 Wo, bo.reshape(E, 1, D))

    out = _run_combine(y, pos0, pos1, w0b, w1b)
    return out.reshape(Bv, Tv, Dv)
